# baseline (speedup 1.0000x reference)
import functools
import math

import jax
import jax.numpy as jnp
from jax import lax
from jax.experimental import pallas as pl
from jax.experimental.pallas import tpu as pltpu
from jax.experimental.pallas import tpu_sc as plsc

VOCAB = 1000000
D = 64
SCALE = math.sqrt(float(D))

NC = 2
NS = 16
NW = NC * NS
L = 16


def _make_gather(B: int, C: int):
    assert B % (NW * C) == 0
    b_per_w = B // NW
    n_chunks = b_per_w // C
    mesh = plsc.VectorSubcoreMesh(core_axis_name="c", subcore_axis_name="s")

    @functools.partial(
        pl.kernel,
        mesh=mesh,
        out_type=jax.ShapeDtypeStruct((B, D), jnp.float32),
        scratch_types=[
            pltpu.VMEM((C,), jnp.int32),
            pltpu.VMEM((C, D), jnp.float32),
            pltpu.SemaphoreType.DMA,
        ],
        compiler_params=pltpu.CompilerParams(use_tc_tiling_on_sc=False),
    )
    def k(idx_hbm, table_hbm, out_hbm, idx_v, rows_v, sem):
        wid = lax.axis_index("s") * NC + lax.axis_index("c")
        base = wid * b_per_w

        def chunk_body(ci, carry):
            off = base + ci * C
            pltpu.sync_copy(idx_hbm.at[pl.ds(off, C)], idx_v)
            pltpu.async_copy(table_hbm.at[idx_v], rows_v, sem).wait()

            def scale_row(i, c2):
                for j in range(D // L):
                    sl = pl.ds(j * L, L)
                    rows_v[i, sl] = rows_v[i, sl] * SCALE
                return c2

            lax.fori_loop(0, C, scale_row, 0)
            pltpu.sync_copy(rows_v, out_hbm.at[pl.ds(off, C)])
            return carry

        lax.fori_loop(0, n_chunks, chunk_body, 0)

    return k


_gather = _make_gather(4096 * 200, 800)


def kernel(x, table):
    idx = x.reshape(-1).astype(jnp.int32)
    out = _gather(idx, table)
    return out.reshape(x.shape[0], x.shape[1], D)


# speedup vs baseline: 1.1121x; 1.1121x over previous
import functools
import math

import jax
import jax.numpy as jnp
from jax import lax
from jax.experimental import pallas as pl
from jax.experimental.pallas import tpu as pltpu
from jax.experimental.pallas import tpu_sc as plsc

VOCAB = 1000000
D = 64
SCALE = math.sqrt(float(D))

NC = 2
NS = 16
NW = NC * NS
L = 16
G = 8


def _make_gather(B: int, C: int):
    assert B % (NW * C) == 0
    b_per_w = B // NW
    n_chunks = b_per_w // C
    assert n_chunks % 2 == 0 and n_chunks >= 4 and C % G == 0
    mesh = plsc.VectorSubcoreMesh(core_axis_name="c", subcore_axis_name="s")

    @functools.partial(
        pl.kernel,
        mesh=mesh,
        out_type=jax.ShapeDtypeStruct((B, D), jnp.float32),
        scratch_types=[
            pltpu.VMEM((b_per_w,), jnp.int32),
            pltpu.VMEM((C, D), jnp.float32),
            pltpu.VMEM((C, D), jnp.float32),
            pltpu.SemaphoreType.DMA,
            pltpu.SemaphoreType.DMA,
            pltpu.SemaphoreType.DMA,
            pltpu.SemaphoreType.DMA,
        ],
        compiler_params=pltpu.CompilerParams(use_tc_tiling_on_sc=False),
    )
    def k(idx_hbm, table_hbm, out_hbm, idx_all, rows0, rows1,
          sg0, sg1, sw0, sw1):
        wid = lax.axis_index("s") * NC + lax.axis_index("c")
        base = wid * b_per_w
        rows = (rows0, rows1)
        sg = (sg0, sg1)
        sw = (sw0, sw1)

        pltpu.sync_copy(idx_hbm.at[pl.ds(base, b_per_w)], idx_all)

        def gather_start(ci, b):
            pltpu.async_copy(
                table_hbm.at[idx_all.at[pl.ds(ci * C, C)]], rows[b], sg[b])

        def gather_wait(b):
            pltpu.make_async_copy(
                out_hbm.at[pl.ds(base, C)], rows[b], sg[b]).wait()

        def wb_start(ci, b):
            pltpu.async_copy(
                rows[b], out_hbm.at[pl.ds(base + ci * C, C)], sw[b])

        def wb_wait(b):
            pltpu.make_async_copy(
                rows[b], out_hbm.at[pl.ds(base, C)], sw[b]).wait()

        def scale(b):
            r = rows[b]

            def scale_grp(g, c2):
                i0 = g * G
                for i in range(G):
                    for j in range(D // L):
                        sl = pl.ds(j * L, L)
                        r[i0 + i, sl] = r[i0 + i, sl] * SCALE
                return c2

            lax.fori_loop(0, C // G, scale_grp, 0)

        def chunk(ci, b, first, last):
            b2 = 1 - b
            gather_wait(b)
            if not first:
                wb_wait(b2)
            if not last:
                gather_start(ci + 1, b2)
            scale(b)
            wb_start(ci, b)

        gather_start(0, 0)
        chunk(0, 0, first=True, last=False)
        chunk(1, 1, first=False, last=False)

        def group(g, carry):
            ci = 2 * g
            chunk(ci, 0, first=False, last=False)
            chunk(ci + 1, 1, first=False, last=False)
            return carry

        lax.fori_loop(1, n_chunks // 2 - 1, group, 0)

        chunk(n_chunks - 2, 0, first=False, last=False)
        chunk(n_chunks - 1, 1, first=False, last=True)
        wb_wait(1)

    return k


_gather = _make_gather(4096 * 200, 800)


def kernel(x, table):
    idx = x.reshape(-1).astype(jnp.int32)
    out = _gather(idx, table)
    return out.reshape(x.shape[0], x.shape[1], D)
